# baseline (speedup 1.0000x reference)
import functools

import jax
import jax.numpy as jnp
from jax import lax
from jax.experimental import pallas as pl
from jax.experimental.pallas import tpu as pltpu
from jax.experimental.pallas import tpu_sc as plsc

_B = 16384
_D = 64

_info = plsc.get_sparse_core_info()
_NC = _info.num_cores
_NS = _info.num_subcores
_NW = _NC * _NS
_BPW = _B // _NW
_CHUNK = 128
_NCHUNK = _BPW // _CHUNK


def _gather_body(ug_t, ig_t, ub_t, ib_t, users, items,
                 ug_o, ig_o, ub_o, ib_o,
                 idxu_v, idxi_v, ug_v, ig_v, ub_v, ib_v, sem):
    wid = lax.axis_index("s") * _NC + lax.axis_index("c")
    base = wid * _BPW
    pltpu.sync_copy(users.at[pl.ds(base, _BPW)], idxu_v)
    pltpu.sync_copy(items.at[pl.ds(base, _BPW)], idxi_v)
    copies = []
    for j in range(_NCHUNK):
        s = pl.ds(j * _CHUNK, _CHUNK)
        copies.append(pltpu.async_copy(ug_t.at[idxu_v.at[s]], ug_v.at[s], sem))
        copies.append(pltpu.async_copy(ig_t.at[idxi_v.at[s]], ig_v.at[s], sem))
        copies.append(pltpu.async_copy(ub_t.at[idxu_v.at[s]], ub_v.at[s], sem))
        copies.append(pltpu.async_copy(ib_t.at[idxi_v.at[s]], ib_v.at[s], sem))
    for c in copies:
        c.wait()
    pltpu.sync_copy(ug_v, ug_o.at[pl.ds(base, _BPW)])
    pltpu.sync_copy(ig_v, ig_o.at[pl.ds(base, _BPW)])
    pltpu.sync_copy(ub_v, ub_o.at[pl.ds(base, _BPW)])
    pltpu.sync_copy(ib_v, ib_o.at[pl.ds(base, _BPW)])


_sc_gather = functools.partial(
    pl.kernel,
    mesh=plsc.VectorSubcoreMesh(core_axis_name="c", subcore_axis_name="s"),
    out_type=[
        jax.ShapeDtypeStruct((_B, _D), jnp.float32),
        jax.ShapeDtypeStruct((_B, _D), jnp.float32),
        jax.ShapeDtypeStruct((_B,), jnp.float32),
        jax.ShapeDtypeStruct((_B,), jnp.float32),
    ],
    scratch_types=[
        pltpu.VMEM((_BPW,), jnp.int32),
        pltpu.VMEM((_BPW,), jnp.int32),
        pltpu.VMEM((_BPW, _D), jnp.float32),
        pltpu.VMEM((_BPW, _D), jnp.float32),
        pltpu.VMEM((_BPW,), jnp.float32),
        pltpu.VMEM((_BPW,), jnp.float32),
        pltpu.SemaphoreType.DMA,
    ],
    compiler_params=pltpu.CompilerParams(use_tc_tiling_on_sc=False),
)(_gather_body)


def _math_body(ug_ref, ig_ref, ub_ref, ib_ref, out_ref):
    ug = ug_ref[...]
    ig = ig_ref[...]
    cu = jnp.maximum(jnp.sqrt(jnp.sum(ug * ug, axis=0, keepdims=True)), 1e-12)
    ci = jnp.maximum(jnp.sqrt(jnp.sum(ig * ig, axis=0, keepdims=True)), 1e-12)
    w = 1.0 / (cu * ci)
    wu = 1.0 / (cu * cu)
    wi = 1.0 / (ci * ci)
    num = jnp.sum(ug * ig * w, axis=1)
    rnu = jnp.sqrt(jnp.sum(ug * ug * wu, axis=1))
    rni = jnp.sqrt(jnp.sum(ig * ig * wi, axis=1))
    den = jnp.maximum(rnu, 1e-8) * jnp.maximum(rni, 1e-8)
    ub = ub_ref[...]
    ib = ib_ref[...]
    nbu = jnp.maximum(jnp.sqrt(jnp.sum(ub * ub)), 1e-12)
    nbi = jnp.maximum(jnp.sqrt(jnp.sum(ib * ib)), 1e-12)
    out_ref[...] = ib / nbi + ub / nbu + num / den


_tc_math = pl.pallas_call(
    _math_body,
    out_shape=jax.ShapeDtypeStruct((_B,), jnp.float32),
)


def kernel(users, items, user_gama, item_gama, user_beta, item_beta):
    users = users.astype(jnp.int32)
    items = items.astype(jnp.int32)
    ub_t = user_beta.reshape(-1)
    ib_t = item_beta.reshape(-1)
    ug, ig, ub, ib = _sc_gather(user_gama, item_gama, ub_t, ib_t, users, items)
    return _tc_math(ug, ig, ub, ib)


# speedup vs baseline: 1.6686x; 1.6686x over previous
import functools

import jax
import jax.numpy as jnp
from jax import lax
from jax.experimental import pallas as pl
from jax.experimental.pallas import tpu as pltpu
from jax.experimental.pallas import tpu_sc as plsc

_B = 16384
_D = 64
_V = 100000

_info = plsc.get_sparse_core_info()
_NC = _info.num_cores
_NS = _info.num_subcores
_NW = _NC * _NS
_DPW = _D // (_NW // 2)
_SEG = 8192
_NSEG = _B // _SEG


def _gather_rows(row_v, idx_v, out_v, write_seg):
    for seg in range(_NSEG):
        def body(g, _):
            iv = idx_v[pl.ds(seg * _SEG + g * 16, 16)]
            out_v[pl.ds(g * 16, 16)] = plsc.load_gather(row_v, [iv])
            return _
        lax.fori_loop(0, _SEG // 16, body, None, unroll=4)
        write_seg(seg)


def _gather_body(ugT_t, igT_t, ub_t, ib_t, users, items,
                 ugT_o, igT_o, ub_o, ib_o,
                 row_v, idx_v, out_v, sem):
    wid = lax.axis_index("s") * _NC + lax.axis_index("c")
    is_user = wid < (_NW // 2)
    local = lax.rem(wid, _NW // 2)
    d_base = local * _DPW

    @pl.when(is_user)
    def _():
        pltpu.sync_copy(users, idx_v)

    @pl.when(jnp.logical_not(is_user))
    def _():
        pltpu.sync_copy(items, idx_v)

    for t in range(_DPW):
        d = d_base + t

        @pl.when(is_user)
        def _():
            pltpu.sync_copy(ugT_t.at[d], row_v)

            def write_seg(seg):
                pltpu.sync_copy(out_v, ugT_o.at[d, pl.ds(seg * _SEG, _SEG)])
            _gather_rows(row_v, idx_v, out_v, write_seg)

        @pl.when(jnp.logical_not(is_user))
        def _():
            pltpu.sync_copy(igT_t.at[d], row_v)

            def write_seg(seg):
                pltpu.sync_copy(out_v, igT_o.at[d, pl.ds(seg * _SEG, _SEG)])
            _gather_rows(row_v, idx_v, out_v, write_seg)

    @pl.when(wid == 0)
    def _():
        pltpu.sync_copy(ub_t, row_v)

        def write_seg(seg):
            pltpu.sync_copy(out_v, ub_o.at[pl.ds(seg * _SEG, _SEG)])
        _gather_rows(row_v, idx_v, out_v, write_seg)

    @pl.when(wid == (_NW // 2))
    def _():
        pltpu.sync_copy(ib_t, row_v)

        def write_seg(seg):
            pltpu.sync_copy(out_v, ib_o.at[pl.ds(seg * _SEG, _SEG)])
        _gather_rows(row_v, idx_v, out_v, write_seg)


_sc_gather = functools.partial(
    pl.kernel,
    mesh=plsc.VectorSubcoreMesh(core_axis_name="c", subcore_axis_name="s"),
    out_type=[
        jax.ShapeDtypeStruct((_D, _B), jnp.float32),
        jax.ShapeDtypeStruct((_D, _B), jnp.float32),
        jax.ShapeDtypeStruct((_B,), jnp.float32),
        jax.ShapeDtypeStruct((_B,), jnp.float32),
    ],
    scratch_types=[
        pltpu.VMEM((_V,), jnp.float32),
        pltpu.VMEM((_B,), jnp.int32),
        pltpu.VMEM((_SEG,), jnp.float32),
        pltpu.SemaphoreType.DMA,
    ],
    compiler_params=pltpu.CompilerParams(needs_layout_passes=False),
)(_gather_body)


def _math_body(ug_ref, ig_ref, ub_ref, ib_ref, out_ref):
    ug = ug_ref[...]
    ig = ig_ref[...]
    cu = jnp.maximum(jnp.sqrt(jnp.sum(ug * ug, axis=1, keepdims=True)), 1e-12)
    ci = jnp.maximum(jnp.sqrt(jnp.sum(ig * ig, axis=1, keepdims=True)), 1e-12)
    w = 1.0 / (cu * ci)
    wu = 1.0 / (cu * cu)
    wi = 1.0 / (ci * ci)
    num = jnp.sum(ug * ig * w, axis=0)
    rnu = jnp.sqrt(jnp.sum(ug * ug * wu, axis=0))
    rni = jnp.sqrt(jnp.sum(ig * ig * wi, axis=0))
    den = jnp.maximum(rnu, 1e-8) * jnp.maximum(rni, 1e-8)
    ub = ub_ref[...]
    ib = ib_ref[...]
    nbu = jnp.maximum(jnp.sqrt(jnp.sum(ub * ub)), 1e-12)
    nbi = jnp.maximum(jnp.sqrt(jnp.sum(ib * ib)), 1e-12)
    out_ref[...] = ib / nbi + ub / nbu + num / den


_tc_math = pl.pallas_call(
    _math_body,
    out_shape=jax.ShapeDtypeStruct((_B,), jnp.float32),
)


def kernel(users, items, user_gama, item_gama, user_beta, item_beta):
    users = users.astype(jnp.int32)
    items = items.astype(jnp.int32)
    ugT_t = user_gama.T
    igT_t = item_gama.T
    ub_t = user_beta.reshape(-1)
    ib_t = item_beta.reshape(-1)
    ugT, igT, ub, ib = _sc_gather(ugT_t, igT_t, ub_t, ib_t, users, items)
    return _tc_math(ugT, igT, ub, ib)


# speedup vs baseline: 2.6083x; 1.5632x over previous
import functools

import jax
import jax.numpy as jnp
from jax import lax
from jax.experimental import pallas as pl
from jax.experimental.pallas import tpu as pltpu
from jax.experimental.pallas import tpu_sc as plsc

_B = 16384
_D = 64
_V = 100000

_info = plsc.get_sparse_core_info()
_NC = _info.num_cores
_NS = _info.num_subcores
_NW = _NC * _NS
_DPW = _D // (_NW // 2)
_SEG = 8192
_NSEG = _B // _SEG


def _gather_rows(row_v, idx_v, out_v, write_seg):
    for seg in range(_NSEG):
        @plsc.parallel_loop(0, _SEG // 16, unroll=8)
        def _(g):
            iv = idx_v[pl.ds(seg * _SEG + g * 16, 16)]
            out_v[pl.ds(g * 16, 16)] = plsc.load_gather(row_v, [iv])
        write_seg(seg)


def _gather_body(ugT_t, igT_t, ub_t, ib_t, users, items,
                 ugT_o, igT_o, ub_o, ib_o,
                 row_v, idx_v, out_v, sem):
    wid = lax.axis_index("s") * _NC + lax.axis_index("c")
    is_user = wid < (_NW // 2)
    local = lax.rem(wid, _NW // 2)
    d_base = local * _DPW

    @pl.when(is_user)
    def _():
        pltpu.sync_copy(users, idx_v)

    @pl.when(jnp.logical_not(is_user))
    def _():
        pltpu.sync_copy(items, idx_v)

    for t in range(_DPW):
        d = d_base + t

        @pl.when(is_user)
        def _():
            pltpu.sync_copy(ugT_t.at[d], row_v)

            def write_seg(seg):
                pltpu.sync_copy(out_v, ugT_o.at[d, pl.ds(seg * _SEG, _SEG)])
            _gather_rows(row_v, idx_v, out_v, write_seg)

        @pl.when(jnp.logical_not(is_user))
        def _():
            pltpu.sync_copy(igT_t.at[d], row_v)

            def write_seg(seg):
                pltpu.sync_copy(out_v, igT_o.at[d, pl.ds(seg * _SEG, _SEG)])
            _gather_rows(row_v, idx_v, out_v, write_seg)

    @pl.when(wid == 0)
    def _():
        pltpu.sync_copy(ub_t, row_v)

        def write_seg(seg):
            pltpu.sync_copy(out_v, ub_o.at[pl.ds(seg * _SEG, _SEG)])
        _gather_rows(row_v, idx_v, out_v, write_seg)

    @pl.when(wid == (_NW // 2) + 1)
    def _():
        pltpu.sync_copy(ib_t, row_v)

        def write_seg(seg):
            pltpu.sync_copy(out_v, ib_o.at[pl.ds(seg * _SEG, _SEG)])
        _gather_rows(row_v, idx_v, out_v, write_seg)


_sc_gather = functools.partial(
    pl.kernel,
    mesh=plsc.VectorSubcoreMesh(core_axis_name="c", subcore_axis_name="s"),
    out_type=[
        jax.ShapeDtypeStruct((_D, _B), jnp.float32),
        jax.ShapeDtypeStruct((_D, _B), jnp.float32),
        jax.ShapeDtypeStruct((_B,), jnp.float32),
        jax.ShapeDtypeStruct((_B,), jnp.float32),
    ],
    scratch_types=[
        pltpu.VMEM((_V,), jnp.float32),
        pltpu.VMEM((_B,), jnp.int32),
        pltpu.VMEM((_SEG,), jnp.float32),
        pltpu.SemaphoreType.DMA,
    ],
    compiler_params=pltpu.CompilerParams(needs_layout_passes=False),
)(_gather_body)


def _math_body(ug_ref, ig_ref, ub_ref, ib_ref, out_ref):
    ug = ug_ref[...]
    ig = ig_ref[...]
    cu = jnp.maximum(jnp.sqrt(jnp.sum(ug * ug, axis=1, keepdims=True)), 1e-12)
    ci = jnp.maximum(jnp.sqrt(jnp.sum(ig * ig, axis=1, keepdims=True)), 1e-12)
    w = 1.0 / (cu * ci)
    wu = 1.0 / (cu * cu)
    wi = 1.0 / (ci * ci)
    num = jnp.sum(ug * ig * w, axis=0)
    rnu = jnp.sqrt(jnp.sum(ug * ug * wu, axis=0))
    rni = jnp.sqrt(jnp.sum(ig * ig * wi, axis=0))
    den = jnp.maximum(rnu, 1e-8) * jnp.maximum(rni, 1e-8)
    ub = ub_ref[...]
    ib = ib_ref[...]
    nbu = jnp.maximum(jnp.sqrt(jnp.sum(ub * ub)), 1e-12)
    nbi = jnp.maximum(jnp.sqrt(jnp.sum(ib * ib)), 1e-12)
    out_ref[...] = ib / nbi + ub / nbu + num / den


_tc_math = pl.pallas_call(
    _math_body,
    out_shape=jax.ShapeDtypeStruct((_B,), jnp.float32),
)


def kernel(users, items, user_gama, item_gama, user_beta, item_beta):
    users = users.astype(jnp.int32)
    items = items.astype(jnp.int32)
    ugT_t = user_gama.T
    igT_t = item_gama.T
    ub_t = user_beta.reshape(-1)
    ib_t = item_beta.reshape(-1)
    ugT, igT, ub, ib = _sc_gather(ugT_t, igT_t, ub_t, ib_t, users, items)
    return _tc_math(ugT, igT, ub, ib)
